# baseline (speedup 1.0000x reference)
import jax
import jax.numpy as jnp
from jax import lax
from jax.experimental import pallas as pl
from jax.experimental.pallas import tpu as pltpu, tpu_sc as plsc

N = 10000
E = 320000
R = 4
D = 128
NUM_BASES = 8

NC = 2
NS = 16
NW = NC * NS
RE = R * E
CH = 48
NB = 6
G = 139
EPW = G * NB * CH
RE_PAD = NW * EPW
ZSLAB = 640
RPT = 624
NM = 3


def _proj_body(x_ref, wb_ref, wrel_ref, y_ref):
    r = pl.program_id(0)
    wr = jnp.zeros((D, D), jnp.float32)
    for b in range(NUM_BASES):
        wr = wr + wrel_ref[r, b] * wb_ref[b]
    y_ref[...] = jnp.dot(x_ref[...], wr, preferred_element_type=jnp.float32)


def _project(X, w_bases, w_rel):
    NBK = 10
    BM = N // NBK
    return pl.pallas_call(
        _proj_body,
        grid=(R, NBK),
        in_specs=[
            pl.BlockSpec((BM, D), lambda r, i: (i, 0)),
            pl.BlockSpec((NUM_BASES, D, D), lambda r, i: (0, 0, 0)),
            pl.BlockSpec(memory_space=pltpu.SMEM),
        ],
        out_specs=pl.BlockSpec((BM, D), lambda r, i: (r * NBK + i, 0)),
        out_shape=jax.ShapeDtypeStruct((R * N, D), jnp.float32),
    )(X, w_bases, w_rel)


def _edge_body(y_hbm, meta_hbm, ew_hbm, zero_hbm, out_hbm,
               acc, rows_v, meta_v, ew_v, sem_m, *sems):
    cid = lax.axis_index("c")
    sid = lax.axis_index("s")
    wid = cid * NS + sid
    sem_g = sems[:NB]
    sem_s = sems[NB:]

    @pl.when(sid < NS - 1)
    def _zmost():
        pltpu.sync_copy(zero_hbm.at[pl.ds(0, RPT)],
                        acc.at[pl.ds(sid * RPT, RPT)])

    @pl.when(sid == NS - 1)
    def _zlast():
        pltpu.sync_copy(zero_hbm, acc.at[pl.ds((NS - 1) * RPT, ZSLAB)])
    plsc.subcore_barrier()

    dnums = lax.GatherDimensionNumbers(
        offset_dims=(), collapsed_slice_dims=(0,), start_index_map=(0,))

    pltpu.async_copy(meta_hbm.at[wid, 0], meta_v.at[0], sem_m)
    pltpu.async_copy(ew_hbm.at[wid, 0], ew_v.at[0], sem_m)

    def group(g, _):
        m = lax.rem(g, NM)
        mnext = lax.rem(g + 1, NM)
        mprev = lax.rem(g + NM - 1, NM)
        pltpu.make_async_copy(meta_hbm.at[wid, g], meta_v.at[m], sem_m).wait()
        pltpu.make_async_copy(ew_hbm.at[wid, g], ew_v.at[m], sem_m).wait()

        @pl.when(g + 1 < G)
        def _prefetch():
            pltpu.async_copy(meta_hbm.at[wid, g + 1], meta_v.at[mnext], sem_m)
            pltpu.async_copy(ew_hbm.at[wid, g + 1], ew_v.at[mnext], sem_m)

        for b in range(NB):
            @pl.when(g > 0)
            def _drain(b=b):
                pltpu.make_async_copy(
                    rows_v.at[b], acc.at[meta_v.at[mprev, 1, b]],
                    sem_s[b]).wait()
            pltpu.async_copy(y_hbm.at[meta_v.at[m, 0, b]], rows_v.at[b],
                             sem_g[b])

        for b in range(NB):
            pltpu.make_async_copy(y_hbm.at[meta_v.at[m, 0, b]], rows_v.at[b],
                                  sem_g[b]).wait()

            def scale(c, __, b=b):
                wv = ew_v[m, b, pl.ds(c * 16, 16)]
                for e in range(16):
                    wbe = lax.gather(wv, jnp.full((16, 1), e, jnp.int32),
                                     dnums, slice_sizes=(1,),
                                     mode=lax.GatherScatterMode.PROMISE_IN_BOUNDS)
                    row = c * 16 + e
                    for gg in range(D // 16):
                        sl = pl.ds(gg * 16, 16)
                        rows_v[b, row, sl] = rows_v[b, row, sl] * wbe
                return __
            lax.fori_loop(0, CH // 16, scale, 0)

            pltpu.async_copy(rows_v.at[b], acc.at[meta_v.at[m, 1, b]],
                             sem_s[b], add=True)
        return _
    lax.fori_loop(0, G, group, 0)

    mlast = (G - 1) % NM
    for b in range(NB):
        pltpu.make_async_copy(rows_v.at[b], acc.at[meta_v.at[mlast, 1, b]],
                              sem_s[b]).wait()
    plsc.subcore_barrier()

    @pl.when(sid < NS - 1)
    def _wmost():
        pltpu.sync_copy(acc.at[pl.ds(sid * RPT, RPT)],
                        out_hbm.at[cid, pl.ds(sid * RPT, RPT)])

    @pl.when(sid == NS - 1)
    def _wlast():
        pltpu.sync_copy(acc.at[pl.ds((NS - 1) * RPT, ZSLAB)],
                        out_hbm.at[cid, pl.ds((NS - 1) * RPT, ZSLAB)])


_edge_kernel = pl.kernel(
    _edge_body,
    out_type=jax.ShapeDtypeStruct((NC, N, D), jnp.float32),
    mesh=plsc.VectorSubcoreMesh(core_axis_name="c", subcore_axis_name="s"),
    scratch_types=[
        pltpu.VMEM_SHARED((N, D), jnp.float32),
        pltpu.VMEM((NB, CH, D), jnp.float32),
        pltpu.VMEM((NM, 2, NB, CH), jnp.int32),
        pltpu.VMEM((NM, NB, CH), jnp.float32),
        pltpu.SemaphoreType.DMA,
    ] + [pltpu.SemaphoreType.DMA] * (2 * NB),
)


def _sum_body(p_ref, o_ref):
    o_ref[...] = p_ref[0] + p_ref[1]


def _sum_partials(p):
    NBK = 10
    BM = N // NBK
    return pl.pallas_call(
        _sum_body,
        grid=(NBK,),
        in_specs=[pl.BlockSpec((NC, BM, D), lambda i: (0, i, 0))],
        out_specs=pl.BlockSpec((BM, D), lambda i: (i, 0)),
        out_shape=jax.ShapeDtypeStruct((N, D), jnp.float32),
    )(p)


def kernel(X, edge_index, edge_weight, w_bases, w_rel):
    ei = edge_index.astype(jnp.int32)
    srcg = (ei[:, 0, :] + (jnp.arange(R, dtype=jnp.int32) * N)[:, None]).reshape(RE)
    dstg = ei[:, 1, :].reshape(RE)
    ewf = edge_weight.reshape(RE).astype(jnp.float32)
    pad = RE_PAD - RE
    srcp = jnp.pad(srcg, (0, pad)).reshape(NW, G, NB, CH)
    dstp = jnp.pad(dstg, (0, pad)).reshape(NW, G, NB, CH)
    ewp = jnp.pad(ewf, (0, pad)).reshape(NW, G, NB, CH)
    meta = jnp.stack([srcp, dstp], axis=2)
    zeros = jnp.zeros((ZSLAB, D), jnp.float32)
    Y = _project(X, w_bases, w_rel)
    partials = _edge_kernel(Y, meta, ewp, zeros)
    return _sum_partials(partials)


# speedup vs baseline: 1.0722x; 1.0722x over previous
import jax
import jax.numpy as jnp
from jax import lax
from jax.experimental import pallas as pl
from jax.experimental.pallas import tpu as pltpu, tpu_sc as plsc

N = 10000
E = 320000
R = 4
D = 128
NUM_BASES = 8

NC = 2
NS = 16
NW = NC * NS
RE = R * E
CH = 64
NB = 5
G = 125
EPW = G * NB * CH
RE_PAD = NW * EPW
ZSLAB = 640
RPT = 624
NM = 3


def _proj_body(x_ref, wb_ref, wrel_ref, y_ref):
    r = pl.program_id(0)
    wr = jnp.zeros((D, D), jnp.float32)
    for b in range(NUM_BASES):
        wr = wr + wrel_ref[r, b] * wb_ref[b]
    y_ref[...] = jnp.dot(x_ref[...], wr, preferred_element_type=jnp.float32)


def _project(X, w_bases, w_rel):
    NBK = 10
    BM = N // NBK
    return pl.pallas_call(
        _proj_body,
        grid=(R, NBK),
        in_specs=[
            pl.BlockSpec((BM, D), lambda r, i: (i, 0)),
            pl.BlockSpec((NUM_BASES, D, D), lambda r, i: (0, 0, 0)),
            pl.BlockSpec(memory_space=pltpu.SMEM),
        ],
        out_specs=pl.BlockSpec((BM, D), lambda r, i: (r * NBK + i, 0)),
        out_shape=jax.ShapeDtypeStruct((R * N, D), jnp.float32),
    )(X, w_bases, w_rel)


def _edge_body(y_hbm, meta_hbm, ew_hbm, zero_hbm, out_hbm,
               acc, rows_v, meta_v, ew_v, sem_m, *sems):
    cid = lax.axis_index("c")
    sid = lax.axis_index("s")
    wid = cid * NS + sid
    sem_g = sems[:NB]
    sem_g2 = sems[NB:2 * NB]
    sem_s = sems[2 * NB:]

    @pl.when(sid < NS - 1)
    def _zmost():
        pltpu.sync_copy(zero_hbm.at[pl.ds(0, RPT)],
                        acc.at[pl.ds(sid * RPT, RPT)])

    @pl.when(sid == NS - 1)
    def _zlast():
        pltpu.sync_copy(zero_hbm, acc.at[pl.ds((NS - 1) * RPT, ZSLAB)])
    plsc.subcore_barrier()

    dnums = lax.GatherDimensionNumbers(
        offset_dims=(), collapsed_slice_dims=(0,), start_index_map=(0,))

    pltpu.async_copy(meta_hbm.at[wid, 0], meta_v.at[0], sem_m)
    pltpu.async_copy(ew_hbm.at[wid, 0], ew_v.at[0], sem_m)

    def group(g, _):
        m = lax.rem(g, NM)
        mnext = lax.rem(g + 1, NM)
        mprev = lax.rem(g + NM - 1, NM)
        pltpu.make_async_copy(meta_hbm.at[wid, g], meta_v.at[m], sem_m).wait()
        pltpu.make_async_copy(ew_hbm.at[wid, g], ew_v.at[m], sem_m).wait()

        @pl.when(g + 1 < G)
        def _prefetch():
            pltpu.async_copy(meta_hbm.at[wid, g + 1], meta_v.at[mnext], sem_m)
            pltpu.async_copy(ew_hbm.at[wid, g + 1], ew_v.at[mnext], sem_m)

        for b in range(NB):
            @pl.when(g > 0)
            def _drain(b=b):
                pltpu.make_async_copy(
                    rows_v.at[b], acc.at[meta_v.at[mprev, 1, b]],
                    sem_s[b]).wait()
            pltpu.async_copy(y_hbm.at[meta_v.at[m, 0, b, pl.ds(0, CH // 2)]],
                             rows_v.at[b, pl.ds(0, CH // 2)], sem_g[b])
            pltpu.async_copy(y_hbm.at[meta_v.at[m, 0, b, pl.ds(CH // 2, CH // 2)]],
                             rows_v.at[b, pl.ds(CH // 2, CH // 2)], sem_g2[b])

        for b in range(NB):
            pltpu.make_async_copy(
                y_hbm.at[meta_v.at[m, 0, b, pl.ds(0, CH // 2)]],
                rows_v.at[b, pl.ds(0, CH // 2)], sem_g[b]).wait()
            pltpu.make_async_copy(
                y_hbm.at[meta_v.at[m, 0, b, pl.ds(CH // 2, CH // 2)]],
                rows_v.at[b, pl.ds(CH // 2, CH // 2)], sem_g2[b]).wait()

            def scale(c, __, b=b):
                wv = ew_v[m, b, pl.ds(c * 16, 16)]
                for e in range(16):
                    wbe = lax.gather(wv, jnp.full((16, 1), e, jnp.int32),
                                     dnums, slice_sizes=(1,),
                                     mode=lax.GatherScatterMode.PROMISE_IN_BOUNDS)
                    row = c * 16 + e
                    for gg in range(D // 16):
                        sl = pl.ds(gg * 16, 16)
                        rows_v[b, row, sl] = rows_v[b, row, sl] * wbe
                return __
            lax.fori_loop(0, CH // 16, scale, 0)

            pltpu.async_copy(rows_v.at[b], acc.at[meta_v.at[m, 1, b]],
                             sem_s[b], add=True)
        return _
    lax.fori_loop(0, G, group, 0)

    mlast = (G - 1) % NM
    for b in range(NB):
        pltpu.make_async_copy(rows_v.at[b], acc.at[meta_v.at[mlast, 1, b]],
                              sem_s[b]).wait()
    plsc.subcore_barrier()

    @pl.when(sid < NS - 1)
    def _wmost():
        pltpu.sync_copy(acc.at[pl.ds(sid * RPT, RPT)],
                        out_hbm.at[cid, pl.ds(sid * RPT, RPT)])

    @pl.when(sid == NS - 1)
    def _wlast():
        pltpu.sync_copy(acc.at[pl.ds((NS - 1) * RPT, ZSLAB)],
                        out_hbm.at[cid, pl.ds((NS - 1) * RPT, ZSLAB)])


_edge_kernel = pl.kernel(
    _edge_body,
    out_type=jax.ShapeDtypeStruct((NC, N, D), jnp.float32),
    mesh=plsc.VectorSubcoreMesh(core_axis_name="c", subcore_axis_name="s"),
    scratch_types=[
        pltpu.VMEM_SHARED((N, D), jnp.float32),
        pltpu.VMEM((NB, CH, D), jnp.float32),
        pltpu.VMEM((NM, 2, NB, CH), jnp.int32),
        pltpu.VMEM((NM, NB, CH), jnp.float32),
        pltpu.SemaphoreType.DMA,
    ] + [pltpu.SemaphoreType.DMA] * (3 * NB),
)


def _sum_body(p_ref, o_ref):
    o_ref[...] = p_ref[0] + p_ref[1]


def _sum_partials(p):
    NBK = 10
    BM = N // NBK
    return pl.pallas_call(
        _sum_body,
        grid=(NBK,),
        in_specs=[pl.BlockSpec((NC, BM, D), lambda i: (0, i, 0))],
        out_specs=pl.BlockSpec((BM, D), lambda i: (i, 0)),
        out_shape=jax.ShapeDtypeStruct((N, D), jnp.float32),
    )(p)


def kernel(X, edge_index, edge_weight, w_bases, w_rel):
    ei = edge_index.astype(jnp.int32)
    srcg = (ei[:, 0, :] + (jnp.arange(R, dtype=jnp.int32) * N)[:, None]).reshape(RE)
    dstg = ei[:, 1, :].reshape(RE)
    ewf = edge_weight.reshape(RE).astype(jnp.float32)
    pad = RE_PAD - RE
    srcp = jnp.pad(srcg, (0, pad)).reshape(NW, G, NB, CH)
    dstp = jnp.pad(dstg, (0, pad)).reshape(NW, G, NB, CH)
    ewp = jnp.pad(ewf, (0, pad)).reshape(NW, G, NB, CH)
    meta = jnp.stack([srcp, dstp], axis=2)
    zeros = jnp.zeros((ZSLAB, D), jnp.float32)
    Y = _project(X, w_bases, w_rel)
    partials = _edge_kernel(Y, meta, ewp, zeros)
    return _sum_partials(partials)


# speedup vs baseline: 1.0917x; 1.0182x over previous
import jax
import jax.numpy as jnp
from jax import lax
from jax.experimental import pallas as pl
from jax.experimental.pallas import tpu as pltpu, tpu_sc as plsc

N = 10000
E = 320000
R = 4
D = 128
NUM_BASES = 8

NC = 2
NS = 16
NW = NC * NS
RE = R * E
CH = 64
NB = 5
G = 125
EPW = G * NB * CH
RE_PAD = NW * EPW
ZSLAB = 640
RPT = 624
NM = 3


def _proj_body(x_ref, wb_ref, wrel_ref, y_ref):
    r = pl.program_id(0)
    wr = jnp.zeros((D, D), jnp.float32)
    for b in range(NUM_BASES):
        wr = wr + wrel_ref[r, b] * wb_ref[b]
    y_ref[...] = jnp.dot(x_ref[...], wr, preferred_element_type=jnp.float32)


def _project(X, w_bases, w_rel):
    NBK = 10
    BM = N // NBK
    return pl.pallas_call(
        _proj_body,
        grid=(R, NBK),
        in_specs=[
            pl.BlockSpec((BM, D), lambda r, i: (i, 0)),
            pl.BlockSpec((NUM_BASES, D, D), lambda r, i: (0, 0, 0)),
            pl.BlockSpec(memory_space=pltpu.SMEM),
        ],
        out_specs=pl.BlockSpec((BM, D), lambda r, i: (r * NBK + i, 0)),
        out_shape=jax.ShapeDtypeStruct((R * N, D), jnp.float32),
    )(X, w_bases, w_rel)


def _edge_body(y_hbm, meta_hbm, ew_hbm, zero_hbm, out_hbm,
               acc, rows_v, meta_v, ew_v, sem_m, *sems):
    cid = lax.axis_index("c")
    sid = lax.axis_index("s")
    wid = cid * NS + sid
    sem_g = sems[:NB]
    sem_s = sems[NB:]

    @pl.when(sid < NS - 1)
    def _zmost():
        pltpu.sync_copy(zero_hbm.at[pl.ds(0, RPT)],
                        acc.at[pl.ds(sid * RPT, RPT)])

    @pl.when(sid == NS - 1)
    def _zlast():
        pltpu.sync_copy(zero_hbm, acc.at[pl.ds((NS - 1) * RPT, ZSLAB)])
    plsc.subcore_barrier()

    dnums = lax.GatherDimensionNumbers(
        offset_dims=(), collapsed_slice_dims=(0,), start_index_map=(0,))

    pltpu.async_copy(meta_hbm.at[wid, 0], meta_v.at[0], sem_m)
    pltpu.async_copy(ew_hbm.at[wid, 0], ew_v.at[0], sem_m)

    def group(g, _):
        m = lax.rem(g, NM)
        mnext = lax.rem(g + 1, NM)
        mprev = lax.rem(g + NM - 1, NM)
        pltpu.make_async_copy(meta_hbm.at[wid, g], meta_v.at[m], sem_m).wait()
        pltpu.make_async_copy(ew_hbm.at[wid, g], ew_v.at[m], sem_m).wait()

        @pl.when(g + 1 < G)
        def _prefetch():
            pltpu.async_copy(meta_hbm.at[wid, g + 1], meta_v.at[mnext], sem_m)
            pltpu.async_copy(ew_hbm.at[wid, g + 1], ew_v.at[mnext], sem_m)

        for b in range(NB):
            @pl.when(g > 0)
            def _drain(b=b):
                pltpu.make_async_copy(
                    rows_v.at[b], acc.at[meta_v.at[mprev, 1, b]],
                    sem_s[b]).wait()
            pltpu.async_copy(y_hbm.at[meta_v.at[m, 0, b]], rows_v.at[b],
                             sem_g[b])

        for b in range(NB):
            pltpu.make_async_copy(y_hbm.at[meta_v.at[m, 0, b]], rows_v.at[b],
                                  sem_g[b]).wait()

            def scale(c, __, b=b):
                wv = ew_v[m, b, pl.ds(c * 16, 16)]
                for e in range(16):
                    wbe = lax.gather(wv, jnp.full((16, 1), e, jnp.int32),
                                     dnums, slice_sizes=(1,),
                                     mode=lax.GatherScatterMode.PROMISE_IN_BOUNDS)
                    row = c * 16 + e
                    for gg in range(D // 16):
                        sl = pl.ds(gg * 16, 16)
                        rows_v[b, row, sl] = rows_v[b, row, sl] * wbe
                return __
            lax.fori_loop(0, CH // 16, scale, 0)

            pltpu.async_copy(rows_v.at[b], acc.at[meta_v.at[m, 1, b]],
                             sem_s[b], add=True)
        return _
    lax.fori_loop(0, G, group, 0)

    mlast = (G - 1) % NM
    for b in range(NB):
        pltpu.make_async_copy(rows_v.at[b], acc.at[meta_v.at[mlast, 1, b]],
                              sem_s[b]).wait()
    plsc.subcore_barrier()

    @pl.when(sid < NS - 1)
    def _wmost():
        pltpu.sync_copy(acc.at[pl.ds(sid * RPT, RPT)],
                        out_hbm.at[cid, pl.ds(sid * RPT, RPT)])

    @pl.when(sid == NS - 1)
    def _wlast():
        pltpu.sync_copy(acc.at[pl.ds((NS - 1) * RPT, ZSLAB)],
                        out_hbm.at[cid, pl.ds((NS - 1) * RPT, ZSLAB)])


_edge_kernel = pl.kernel(
    _edge_body,
    out_type=jax.ShapeDtypeStruct((NC, N, D), jnp.float32),
    mesh=plsc.VectorSubcoreMesh(core_axis_name="c", subcore_axis_name="s"),
    scratch_types=[
        pltpu.VMEM_SHARED((N, D), jnp.float32),
        pltpu.VMEM((NB, CH, D), jnp.float32),
        pltpu.VMEM((NM, 2, NB, CH), jnp.int32),
        pltpu.VMEM((NM, NB, CH), jnp.float32),
        pltpu.SemaphoreType.DMA,
    ] + [pltpu.SemaphoreType.DMA] * (2 * NB),
)


def _sum_body(p_ref, o_ref):
    o_ref[...] = p_ref[0] + p_ref[1]


def _sum_partials(p):
    NBK = 10
    BM = N // NBK
    return pl.pallas_call(
        _sum_body,
        grid=(NBK,),
        in_specs=[pl.BlockSpec((NC, BM, D), lambda i: (0, i, 0))],
        out_specs=pl.BlockSpec((BM, D), lambda i: (i, 0)),
        out_shape=jax.ShapeDtypeStruct((N, D), jnp.float32),
    )(p)


def kernel(X, edge_index, edge_weight, w_bases, w_rel):
    ei = edge_index.astype(jnp.int32)
    srcg = (ei[:, 0, :] + (jnp.arange(R, dtype=jnp.int32) * N)[:, None]).reshape(RE)
    dstg = ei[:, 1, :].reshape(RE)
    ewf = edge_weight.reshape(RE).astype(jnp.float32)
    pad = RE_PAD - RE
    srcp = jnp.pad(srcg, (0, pad)).reshape(NW, G, NB, CH)
    dstp = jnp.pad(dstg, (0, pad)).reshape(NW, G, NB, CH)
    ewp = jnp.pad(ewf, (0, pad)).reshape(NW, G, NB, CH)
    meta = jnp.stack([srcp, dstp], axis=2)
    zeros = jnp.zeros((ZSLAB, D), jnp.float32)
    Y = _project(X, w_bases, w_rel)
    partials = _edge_kernel(Y, meta, ewp, zeros)
    return _sum_partials(partials)
